# Initial kernel scaffold; baseline (speedup 1.0000x reference)
import functools

import jax
import jax.numpy as jnp
from jax import lax
from jax.experimental import pallas as pl
from jax.experimental.pallas import tpu as pltpu
from jax.experimental.pallas import tpu_sc as plsc

_N = 16384
_F = 39
_D = 13
_S = 26
_V = 100000
_EMB = 16
_EPS = 1e-5

_NC = 2
_NS = 16
_NW = _NC * _NS
_CHUNK = _N // _NW
_B = 64
_NB = _CHUNK // _B
_ROWS = _B * _S
_IDXW = 128
_NDMA = _ROWS // _IDXW


def _fold_body(wd1, bd1, g1, be1, wd2, bd2, g2, be2, weff, cvec):
    s = (1.0 + _EPS) ** -0.5
    s2 = 1.0 / (1.0 + _EPS)
    u = jnp.dot(g2[...], wd2[...], preferred_element_type=jnp.float32)
    v = g1[...] * u
    w1 = jnp.dot(v, wd1[...], preferred_element_type=jnp.float32)
    weff[...] = s2 * w1
    c = (s2 * jnp.sum(v * bd1[...]) + s * jnp.sum(u * be1[...])
         + s * jnp.sum(g2[...] * bd2[...]) + jnp.sum(be2[...]))
    cvec[...] = jnp.full((1, 16), c, jnp.float32)


def _fold(Wd1, bd1, g1, be1, Wd2, bd2, g2, be2):
    weff, cvec = pl.pallas_call(
        _fold_body,
        out_shape=[
            jax.ShapeDtypeStruct((1, _F * _EMB), jnp.float32),
            jax.ShapeDtypeStruct((1, 16), jnp.float32),
        ],
    )(Wd1, bd1.reshape(1, -1), g1.reshape(1, -1), be1.reshape(1, -1),
      Wd2, bd2.reshape(1, -1), g2.reshape(1, -1), be2.reshape(1, -1))
    return weff.reshape(_F, _EMB), cvec.reshape(16)


def _sc_body(idx_h, xd_h, xv_h, w1_h, b1_h, w2_h, b2_h, weff_h, c_h,
             e1_h, e2_h, out_h,
             w1_v, b1_v, w2_v, b2_v, weff_v, c_v,
             idx_v, e1_v, e2_v, xv_v, xd_v, tot_v, sem1, sem2):
    wid = lax.axis_index("s") * _NC + lax.axis_index("c")
    base = wid * _CHUNK
    pltpu.sync_copy(w1_h, w1_v)
    pltpu.sync_copy(b1_h, b1_v)
    pltpu.sync_copy(w2_h, w2_v)
    pltpu.sync_copy(b2_h, b2_v)
    pltpu.sync_copy(weff_h, weff_v)
    pltpu.sync_copy(c_h, c_v)

    def blk_body(blk, carry):
        idxrow0 = wid * (_CHUNK * _S // _IDXW) + blk * _NDMA
        ex0 = base + blk * _B
        pltpu.sync_copy(idx_h.at[pl.ds(idxrow0, _NDMA), :], idx_v)
        copies = []
        for j in range(_NDMA):
            copies.append(pltpu.async_copy(
                e1_h.at[idx_v.at[j]], e1_v.at[pl.ds(j * _IDXW, _IDXW)], sem1))
            copies.append(pltpu.async_copy(
                e2_h.at[idx_v.at[j]], e2_v.at[pl.ds(j * _IDXW, _IDXW)], sem2))
        pltpu.sync_copy(xv_h.at[pl.ds(ex0, _B), :], xv_v)
        pltpu.sync_copy(xd_h.at[pl.ds(ex0, _B), :], xd_v)
        for cp in copies:
            cp.wait()

        def ex_body(i, carry2):
            rb = i * _S
            acc_f = jnp.zeros((16,), jnp.float32)
            acc_s = jnp.zeros((16,), jnp.float32)
            acc_q = jnp.zeros((16,), jnp.float32)
            acc_d = jnp.zeros((16,), jnp.float32)
            for d in range(_D):
                x = xd_v[i, d]
                xw = xv_v[i, d]
                r1 = w1_v[d, :] * x + b1_v[d, :]
                acc_f = acc_f + r1 * xw
                a = (w2_v[d, :] * x + b2_v[d, :]) * xw
                acc_s = acc_s + a
                acc_q = acc_q + a * a
                acc_d = acc_d + a * weff_v[d, :]
            for sf in range(_S):
                xw = xv_v[i, _D + sf]
                acc_f = acc_f + e1_v[rb + sf, :] * xw
                a = e2_v[rb + sf, :] * xw
                acc_s = acc_s + a
                acc_q = acc_q + a * a
                acc_d = acc_d + a * weff_v[_D + sf, :]
            tot = acc_f + acc_d + 0.5 * (acc_s * acc_s - acc_q)
            tot_v[i] = jnp.sum(tot, axis=0)
            return carry2

        lax.fori_loop(0, _B, ex_body, 0)

        cv = c_v[...]
        for j in range(_B // 16):
            t = tot_v[pl.ds(j * 16, 16)] + cv
            tot_v[pl.ds(j * 16, 16)] = 1.0 / (1.0 + jnp.exp(-t))
        pltpu.sync_copy(tot_v, out_h.at[pl.ds(ex0, _B)])
        return carry

    lax.fori_loop(0, _NB, blk_body, 0)


_sc_kernel = functools.partial(
    pl.kernel,
    out_type=jax.ShapeDtypeStruct((_N,), jnp.float32),
    mesh=plsc.VectorSubcoreMesh(core_axis_name="c", subcore_axis_name="s",
                                num_cores=_NC, num_subcores=_NS),
    scratch_types=[
        pltpu.VMEM((_D, _EMB), jnp.float32),
        pltpu.VMEM((_D, _EMB), jnp.float32),
        pltpu.VMEM((_D, _EMB), jnp.float32),
        pltpu.VMEM((_D, _EMB), jnp.float32),
        pltpu.VMEM((_F, _EMB), jnp.float32),
        pltpu.VMEM((16,), jnp.float32),
        pltpu.VMEM((_NDMA, _IDXW), jnp.int32),
        pltpu.VMEM((_ROWS, _EMB), jnp.float32),
        pltpu.VMEM((_ROWS, _EMB), jnp.float32),
        pltpu.VMEM((_B, _F), jnp.float32),
        pltpu.VMEM((_B, _D), jnp.float32),
        pltpu.VMEM((_B,), jnp.float32),
        pltpu.SemaphoreType.DMA,
        pltpu.SemaphoreType.DMA,
    ],
)(_sc_body)


def kernel(Xi, Xv, W1d, b1d, E1, W2d, b2d, E2,
           Wd1, bd1, g1, be1, Wd2, bd2, g2, be2):
    idx = Xi[:, _D:, 0].astype(jnp.int32)
    flat_idx = (idx + (jnp.arange(_S, dtype=jnp.int32) * _V)[None, :])
    flat_idx = flat_idx.reshape(_N * _S // _IDXW, _IDXW)
    xd = Xi[:, :_D, 0].astype(jnp.float32)
    weff, cvec = _fold(Wd1, bd1, g1, be1, Wd2, bd2, g2, be2)
    out = _sc_kernel(flat_idx, xd, Xv, W1d, b1d, W2d, b2d, weff, cvec,
                     E1.reshape(_S * _V, _EMB), E2.reshape(_S * _V, _EMB))
    return out


# speedup vs baseline: 5.6597x; 5.6597x over previous
import functools

import jax
import jax.numpy as jnp
from jax import lax
from jax.experimental import pallas as pl
from jax.experimental.pallas import tpu as pltpu
from jax.experimental.pallas import tpu_sc as plsc

_N = 16384
_F = 39
_D = 13
_S = 26
_V = 100000
_EMB = 16
_EPS = 1e-5

_NC = 2
_NS = 16
_NW = _NC * _NS
_CHUNK = _N // _NW
_B = 64
_NB = _CHUNK // _B
_ROWS = _B * _S
_IDXW = 128
_NDMA = _ROWS // _IDXW


def _fold_body(wd1, bd1, g1, be1, wd2, bd2, g2, be2, weff, cvec):
    s = (1.0 + _EPS) ** -0.5
    s2 = 1.0 / (1.0 + _EPS)
    u = jnp.dot(g2[...], wd2[...], preferred_element_type=jnp.float32)
    v = g1[...] * u
    w1 = jnp.dot(v, wd1[...], preferred_element_type=jnp.float32)
    weff[...] = s2 * w1
    c = (s2 * jnp.sum(v * bd1[...]) + s * jnp.sum(u * be1[...])
         + s * jnp.sum(g2[...] * bd2[...]) + jnp.sum(be2[...]))
    cvec[...] = jnp.full((1, 16), c, jnp.float32)


def _fold(Wd1, bd1, g1, be1, Wd2, bd2, g2, be2):
    weff, cvec = pl.pallas_call(
        _fold_body,
        out_shape=[
            jax.ShapeDtypeStruct((1, _F * _EMB), jnp.float32),
            jax.ShapeDtypeStruct((1, 16), jnp.float32),
        ],
    )(Wd1, bd1.reshape(1, -1), g1.reshape(1, -1), be1.reshape(1, -1),
      Wd2, bd2.reshape(1, -1), g2.reshape(1, -1), be2.reshape(1, -1))
    return weff.reshape(_F, _EMB), cvec.reshape(16)


def _lanesum(v, lane):
    dnums = lax.GatherDimensionNumbers(
        offset_dims=(), collapsed_slice_dims=(0,), start_index_map=(0,))
    for sh in (8, 4, 2, 1):
        perm = jnp.bitwise_xor(lane, sh)
        v = v + lax.gather(v, perm[:, None], dnums, (1,),
                           mode=lax.GatherScatterMode.PROMISE_IN_BOUNDS)
    return v


def _sc_body(idx_h, xd_h, xv_h, w1_h, b1_h, w2_h, b2_h, weff_h, c_h,
             e1_h, e2_h, out_h,
             w1_v, b1_v, w2_v, b2_v, weff_v, c_v,
             idx_v, e1_v, e2_v, xv_v, xd_v, tot_v, sem1, sem2):
    wid = lax.axis_index("s") * _NC + lax.axis_index("c")
    base = wid * _CHUNK
    pltpu.sync_copy(w1_h, w1_v)
    pltpu.sync_copy(b1_h, b1_v)
    pltpu.sync_copy(w2_h, w2_v)
    pltpu.sync_copy(b2_h, b2_v)
    pltpu.sync_copy(weff_h, weff_v)
    pltpu.sync_copy(c_h, c_v)

    lane = lax.iota(jnp.int32, 16)

    def blk_body(blk, carry):
        row0 = wid * (_CHUNK * _S) + blk * _ROWS
        ex0 = base + blk * _B
        pltpu.sync_copy(idx_h.at[pl.ds(row0, _ROWS)], idx_v)
        copies = []
        for j in range(_NDMA):
            sl = pl.ds(j * _IDXW, _IDXW)
            copies.append(pltpu.async_copy(
                e1_h.at[idx_v.at[sl]], e1_v.at[sl], sem1))
            copies.append(pltpu.async_copy(
                e2_h.at[idx_v.at[sl]], e2_v.at[sl], sem2))
        pltpu.sync_copy(xv_h.at[pl.ds(ex0, _B), :], xv_v)
        pltpu.sync_copy(xd_h.at[pl.ds(ex0, _B), :], xd_v)
        for cp in copies:
            cp.wait()

        cv = c_v[...]

        def grp_body(g, carry2):
            def ex_body(k, totvec):
                i = g * 16 + k
                rb = i * _S
                xw0 = xv_v[i, pl.ds(0, 16)]
                xw1 = xv_v[i, pl.ds(16, 16)]
                xw2 = xv_v[i, pl.ds(32, 16)]
                xws = (xw0, xw1, xw2)
                xdv = xd_v[i, pl.ds(0, 16)]
                acc_f = jnp.zeros((16,), jnp.float32)
                acc_s = jnp.zeros((16,), jnp.float32)
                acc_q = jnp.zeros((16,), jnp.float32)
                acc_d = jnp.zeros((16,), jnp.float32)
                for d in range(_D):
                    x = xdv[d]
                    xw = xws[d // 16][d % 16]
                    r1 = w1_v[d, :] * x + b1_v[d, :]
                    acc_f = acc_f + r1 * xw
                    a = (w2_v[d, :] * x + b2_v[d, :]) * xw
                    acc_s = acc_s + a
                    acc_q = acc_q + a * a
                    acc_d = acc_d + a * weff_v[d, :]
                for sf in range(_S):
                    f = _D + sf
                    xw = xws[f // 16][f % 16]
                    acc_f = acc_f + e1_v[rb + sf, :] * xw
                    a = e2_v[rb + sf, :] * xw
                    acc_s = acc_s + a
                    acc_q = acc_q + a * a
                    acc_d = acc_d + a * weff_v[f, :]
                tot = acc_f + acc_d + 0.5 * (acc_s * acc_s - acc_q)
                t = _lanesum(tot, lane)
                return jnp.where(lane == k, t, totvec)

            totvec = lax.fori_loop(0, 16, ex_body, jnp.zeros((16,), jnp.float32))
            sig = 1.0 / (1.0 + jnp.exp(-(totvec + cv)))
            tot_v[pl.ds(g * 16, 16)] = sig
            return carry2

        lax.fori_loop(0, _B // 16, grp_body, 0)
        pltpu.sync_copy(tot_v, out_h.at[pl.ds(ex0, _B)])
        return carry

    lax.fori_loop(0, _NB, blk_body, 0)


_sc_kernel = functools.partial(
    pl.kernel,
    out_type=jax.ShapeDtypeStruct((_N,), jnp.float32),
    mesh=plsc.VectorSubcoreMesh(core_axis_name="c", subcore_axis_name="s",
                                num_cores=_NC, num_subcores=_NS),
    compiler_params=pltpu.CompilerParams(use_tc_tiling_on_sc=False),
    scratch_types=[
        pltpu.VMEM((_D, _EMB), jnp.float32),
        pltpu.VMEM((_D, _EMB), jnp.float32),
        pltpu.VMEM((_D, _EMB), jnp.float32),
        pltpu.VMEM((_D, _EMB), jnp.float32),
        pltpu.VMEM((_F, _EMB), jnp.float32),
        pltpu.VMEM((16,), jnp.float32),
        pltpu.VMEM((_ROWS,), jnp.int32),
        pltpu.VMEM((_ROWS, _EMB), jnp.float32),
        pltpu.VMEM((_ROWS, _EMB), jnp.float32),
        pltpu.VMEM((_B, 48), jnp.float32),
        pltpu.VMEM((_B, 16), jnp.float32),
        pltpu.VMEM((_B,), jnp.float32),
        pltpu.SemaphoreType.DMA,
        pltpu.SemaphoreType.DMA,
    ],
)(_sc_body)


def kernel(Xi, Xv, W1d, b1d, E1, W2d, b2d, E2,
           Wd1, bd1, g1, be1, Wd2, bd2, g2, be2):
    idx = Xi[:, _D:, 0].astype(jnp.int32)
    flat_idx = (idx + (jnp.arange(_S, dtype=jnp.int32) * _V)[None, :])
    flat_idx = flat_idx.reshape(_N * _S)
    xd = jnp.pad(Xi[:, :_D, 0].astype(jnp.float32), ((0, 0), (0, 16 - _D)))
    xvp = jnp.pad(Xv, ((0, 0), (0, 48 - _F)))
    weff, cvec = _fold(Wd1, bd1, g1, be1, Wd2, bd2, g2, be2)
    out = _sc_kernel(flat_idx, xd, xvp, W1d, b1d, W2d, b2d, weff, cvec,
                     E1.reshape(_S * _V, _EMB), E2.reshape(_S * _V, _EMB))
    return out
